# Initial kernel scaffold; baseline (speedup 1.0000x reference)
import jax
import jax.numpy as jnp
from jax.experimental import pallas as pl


def kernel(adj, features, attn_kernel):
    raise NotImplementedError("write your pallas kernel here")


# speedup vs baseline: 27.5972x; 27.5972x over previous
import functools

import jax
import jax.numpy as jnp
from jax import lax
from jax.experimental import pallas as pl
from jax.experimental.pallas import tpu as pltpu
from jax.experimental.pallas import tpu_sc as plsc

KNB = 32
NB, NN, FF = 2, 4096, 768
P0 = 64
NEG = -1000000000.0
NC, NS = 2, 16
NW = NC * NS
RPW = NB * NN // NW


RB = 512


def _imp_body(feat_ref, ak_ref, imp_ref):
    f = feat_ref[0]
    ak = ak_ref[...]
    imp_col = jnp.dot(f, ak, preferred_element_type=jnp.float32)
    imp_ref[0, 0, :] = imp_col[:, 0]


def _imp(features, attn_kernel):
    return pl.pallas_call(
        _imp_body,
        grid=(NB,),
        in_specs=[
            pl.BlockSpec((1, NN, FF), lambda b: (b, 0, 0)),
            pl.BlockSpec((FF, 1), lambda b: (0, 0)),
        ],
        out_specs=pl.BlockSpec((1, 1, NN), lambda b: (b, 0, 0)),
        out_shape=jax.ShapeDtypeStruct((NB, 1, NN), jnp.float32),
    )(features, attn_kernel)


def _rank_body(ic_ref, ir_ref, rank_ref):
    t = pl.program_id(1)
    u = pl.program_id(2)
    col = ic_ref[0]
    row = ir_ref[0]
    jc = lax.broadcasted_iota(jnp.int32, (RB, 1), 0) + t * RB
    jp = lax.broadcasted_iota(jnp.int32, (1, RB), 1) + u * RB
    hit = (row > col) | ((row == col) & (jp < jc))
    cnt = jnp.sum(hit.astype(jnp.int32), axis=1)

    @pl.when(u == 0)
    def _init():
        rank_ref[0, 0, :] = cnt

    @pl.when(u != 0)
    def _acc():
        rank_ref[0, 0, :] = rank_ref[0, 0, :] + cnt


def _rank(imp_col3, imp_row3):
    return pl.pallas_call(
        _rank_body,
        grid=(NB, NN // RB, NN // RB),
        in_specs=[
            pl.BlockSpec((1, RB, 1), lambda b, t, u: (b, t, 0)),
            pl.BlockSpec((1, 1, RB), lambda b, t, u: (b, 0, u)),
        ],
        out_specs=pl.BlockSpec((1, 1, RB), lambda b, t, u: (b, 0, t)),
        out_shape=jax.ShapeDtypeStruct((NB, 1, NN), jnp.int32),
    )(imp_col3, imp_row3)


def _sc_select_body(adj_hbm, imp_hbm, rank_hbm, w_hbm, i_hbm,
                    rank_v, imp_v, order_v, svals_v,
                    idx0_v, idx1_v, g0_v, g1_v, arow_v,
                    selv_v, seli_v, wbuf_v, ibuf_v,
                    sem0, sem1):
    cid = lax.axis_index("c")
    sid = lax.axis_index("s")
    b = cid
    i0 = sid * RPW

    pltpu.sync_copy(rank_hbm.at[b], rank_v)
    pltpu.sync_copy(imp_hbm.at[b], imp_v)

    def build(t, carry):
        off = pl.ds(t * 16, 16)
        r16 = rank_v[off]
        plsc.store_scatter(order_v, [r16], lax.iota(jnp.int32, 16) + t * 16)
        plsc.store_scatter(svals_v, [r16], imp_v[off])
        return carry
    lax.fori_loop(0, NN // 16, build, 0)

    zero16 = jnp.zeros((16,), jnp.int32)
    neg16 = jnp.full((16,), NEG, jnp.float32)

    def issue(row, idx_ref, g_ref, sem):
        ib = row * NN
        for c in range(P0 // 16):
            idx_ref[pl.ds(c * 16, 16)] = order_v[pl.ds(c * 16, 16)] + ib
        pltpu.async_copy(adj_hbm.at[idx_ref], g_ref, sem)

    def process(row, k, idx_ref, g_ref, sem):
        pltpu.make_async_copy(adj_hbm.at[idx_ref], g_ref, sem).wait()
        seli_v[pl.ds(0, 16)] = zero16
        seli_v[pl.ds(16, 16)] = zero16
        selv_v[pl.ds(0, 16)] = neg16
        selv_v[pl.ds(16, 16)] = neg16
        cnt = jnp.zeros((16,), jnp.int32)
        for c in range(P0 // 16):
            off = pl.ds(c * 16, 16)
            m = g_ref[off] > 0.0
            pos = cnt + plsc.cumsum(m.astype(jnp.int32)) - 1
            wm = m & (pos < KNB)
            plsc.store_scatter(seli_v, [pos], order_v[off], mask=wm)
            plsc.store_scatter(selv_v, [pos], svals_v[off], mask=wm)
            cnt = cnt + plsc.all_reduce_population_count(m)

        @pl.when(jnp.max(cnt) < KNB)
        def _fallback():
            rb = pl.multiple_of(row * NN, 8)
            pltpu.sync_copy(adj_hbm.at[pl.ds(rb, NN)], arow_v)
            seli_v[pl.ds(0, 16)] = zero16
            seli_v[pl.ds(16, 16)] = zero16
            selv_v[pl.ds(0, 16)] = neg16
            selv_v[pl.ds(16, 16)] = neg16

            def fb(t, cnt2):
                off = pl.ds(t * 16, 16)
                o16 = order_v[off]
                m2 = plsc.load_gather(arow_v, [o16]) > 0.0
                pos2 = cnt2 + plsc.cumsum(m2.astype(jnp.int32)) - 1
                wm2 = m2 & (pos2 < KNB)
                plsc.store_scatter(seli_v, [pos2], o16, mask=wm2)
                plsc.store_scatter(selv_v, [pos2], svals_v[off], mask=wm2)
                return cnt2 + plsc.all_reduce_population_count(m2)
            lax.fori_loop(0, NN // 16, fb, jnp.zeros((16,), jnp.int32))

        v0 = selv_v[pl.ds(0, 16)]
        v1 = selv_v[pl.ds(16, 16)]
        mxv = jnp.broadcast_to(jnp.maximum(jnp.max(v0), jnp.max(v1)), (16,))
        e0 = jnp.exp(v0 - mxv)
        e1 = jnp.exp(v1 - mxv)
        sv = jnp.broadcast_to(jnp.sum(e0) + jnp.sum(e1), (16,))
        wbuf_v[k, pl.ds(0, 16)] = e0 / sv
        wbuf_v[k, pl.ds(16, 16)] = e1 / sv
        ibuf_v[k, pl.ds(0, 16)] = seli_v[pl.ds(0, 16)]
        ibuf_v[k, pl.ds(16, 16)] = seli_v[pl.ds(16, 16)]

    issue(i0, idx0_v, g0_v, sem0)

    def pair(kk, carry):
        ka = 2 * kk
        kb = 2 * kk + 1
        issue(i0 + kb, idx1_v, g1_v, sem1)
        process(i0 + ka, ka, idx0_v, g0_v, sem0)

        @pl.when(kk < RPW // 2 - 1)
        def _next():
            issue(i0 + kb + 1, idx0_v, g0_v, sem0)
        process(i0 + kb, kb, idx1_v, g1_v, sem1)
        return carry
    lax.fori_loop(0, RPW // 2, pair, 0)

    pltpu.sync_copy(wbuf_v, w_hbm.at[b, pl.ds(i0, RPW)])
    pltpu.sync_copy(ibuf_v, i_hbm.at[b, pl.ds(i0, RPW)])


def _sc_select(adj_flat, imp, rank):
    mesh = plsc.VectorSubcoreMesh(
        core_axis_name="c", subcore_axis_name="s",
        num_cores=NC, num_subcores=NS)
    fn = pl.kernel(
        _sc_select_body,
        out_type=(
            jax.ShapeDtypeStruct((NB, NN, KNB), jnp.float32),
            jax.ShapeDtypeStruct((NB, NN, KNB), jnp.int32),
        ),
        mesh=mesh,
        compiler_params=pltpu.CompilerParams(needs_layout_passes=False),
        scratch_types=[
            pltpu.VMEM((NN,), jnp.int32),
            pltpu.VMEM((NN,), jnp.float32),
            pltpu.VMEM((NN,), jnp.int32),
            pltpu.VMEM((NN,), jnp.float32),
            pltpu.VMEM((P0,), jnp.int32),
            pltpu.VMEM((P0,), jnp.int32),
            pltpu.VMEM((P0,), jnp.float32),
            pltpu.VMEM((P0,), jnp.float32),
            pltpu.VMEM((NN,), jnp.float32),
            pltpu.VMEM((KNB,), jnp.float32),
            pltpu.VMEM((KNB,), jnp.int32),
            pltpu.VMEM((RPW, KNB), jnp.float32),
            pltpu.VMEM((RPW, KNB), jnp.int32),
            pltpu.SemaphoreType.DMA,
            pltpu.SemaphoreType.DMA,
        ],
    )
    return fn(adj_flat, imp, rank)


def kernel(adj, features, attn_kernel):
    imp = _imp(features, attn_kernel)
    rank = _rank(imp.reshape(NB, NN, 1), imp)
    top_w, top_i = _sc_select(adj.reshape(NN * NN),
                              imp.reshape(NB, NN), rank.reshape(NB, NN))
    return (top_w, top_i)


# speedup vs baseline: 36.5415x; 1.3241x over previous
import functools

import jax
import jax.numpy as jnp
from jax import lax
from jax.experimental import pallas as pl
from jax.experimental.pallas import tpu as pltpu
from jax.experimental.pallas import tpu_sc as plsc

KNB = 32
NB, NN, FF = 2, 4096, 768
P0 = 64
NEG = -1000000000.0
NC, NS = 2, 16
NW = NC * NS
RPW = NB * NN // NW


RB = 512


def _imp_body(feat_ref, ak_ref, imp_ref):
    f = feat_ref[0]
    ak = ak_ref[...]
    imp_col = jnp.dot(f, ak, preferred_element_type=jnp.float32)
    imp_ref[0, 0, :] = imp_col[:, 0]


def _imp(features, attn_kernel):
    return pl.pallas_call(
        _imp_body,
        grid=(NB,),
        in_specs=[
            pl.BlockSpec((1, NN, FF), lambda b: (b, 0, 0)),
            pl.BlockSpec((FF, 1), lambda b: (0, 0)),
        ],
        out_specs=pl.BlockSpec((1, 1, NN), lambda b: (b, 0, 0)),
        out_shape=jax.ShapeDtypeStruct((NB, 1, NN), jnp.float32),
    )(features, attn_kernel)


def _rank_body(ic_ref, ir_ref, rank_ref, acc_ref):
    t = pl.program_id(1)
    u = pl.program_id(2)
    col = ic_ref[0]
    row = ir_ref[0]

    ones = jnp.ones((RB, 1), jnp.float32)

    def put(hit):
        cnt = jnp.dot(hit.astype(jnp.float32), ones,
                      preferred_element_type=jnp.float32)[:, 0]

        @pl.when(u == 0)
        def _init():
            acc_ref[...] = cnt

        @pl.when(u != 0)
        def _acc():
            acc_ref[...] = acc_ref[...] + cnt

    @pl.when(u == t)
    def _d():
        jc = lax.broadcasted_iota(jnp.int32, (RB, 1), 0)
        jp = lax.broadcasted_iota(jnp.int32, (1, RB), 1)
        put((row > col) | ((row == col) & (jp < jc)))

    @pl.when(u < t)
    def _lo():
        put(row >= col)

    @pl.when(u > t)
    def _hi():
        put(row > col)

    @pl.when(u == NN // RB - 1)
    def _emit():
        rank_ref[0, 0, :] = acc_ref[...].astype(jnp.int32)


def _rank(imp_col3, imp_row3):
    return pl.pallas_call(
        _rank_body,
        grid=(NB, NN // RB, NN // RB),
        in_specs=[
            pl.BlockSpec((1, RB, 1), lambda b, t, u: (b, t, 0)),
            pl.BlockSpec((1, 1, RB), lambda b, t, u: (b, 0, u)),
        ],
        out_specs=pl.BlockSpec((1, 1, RB), lambda b, t, u: (b, 0, t)),
        out_shape=jax.ShapeDtypeStruct((NB, 1, NN), jnp.int32),
        scratch_shapes=[pltpu.VMEM((RB,), jnp.float32)],
    )(imp_col3, imp_row3)


NRING = 4
PAIRS = RPW // 2


def _sc_select_body(adj_hbm, imp_hbm, rank_hbm, w_hbm, i_hbm,
                    rank_v, imp_v, order_v, svals_v,
                    idx_bufs, g_bufs, arow_v,
                    selv_v, seli_v, wbuf_v, ibuf_v, sems):
    cid = lax.axis_index("c")
    sid = lax.axis_index("s")
    b = cid
    i0 = sid * RPW

    pltpu.sync_copy(rank_hbm.at[b], rank_v)
    pltpu.sync_copy(imp_hbm.at[b], imp_v)

    def build(t, carry):
        off = pl.ds(t * 16, 16)
        r16 = rank_v[off]
        plsc.store_scatter(order_v, [r16], lax.iota(jnp.int32, 16) + t * 16)
        plsc.store_scatter(svals_v, [r16], imp_v[off])
        return carry
    lax.fori_loop(0, NN // 16, build, 0)

    zero16 = jnp.zeros((16,), jnp.int32)
    neg16 = jnp.full((16,), NEG, jnp.float32)

    def issue(pq, r):
        idx_ref, g_ref, sem = idx_bufs[r], g_bufs[r], sems[r]
        for h in range(2):
            ib = (i0 + 2 * pq + h) * NN
            for c in range(P0 // 16):
                idx_ref[pl.ds(h * P0 + c * 16, 16)] = (
                    order_v[pl.ds(c * 16, 16)] + ib)
        pltpu.async_copy(adj_hbm.at[idx_ref], g_ref, sem)

    def process(row, k, goff, idx_ref, g_ref):
        seli_v[pl.ds(0, 16)] = zero16
        seli_v[pl.ds(16, 16)] = zero16
        selv_v[pl.ds(0, 16)] = neg16
        selv_v[pl.ds(16, 16)] = neg16
        cnt = jnp.zeros((16,), jnp.int32)
        for c in range(P0 // 16):
            off = pl.ds(c * 16, 16)
            m = g_ref[pl.ds(goff + c * 16, 16)] > 0.0
            pos = cnt + plsc.cumsum(m.astype(jnp.int32)) - 1
            wm = m & (pos < KNB)
            plsc.store_scatter(seli_v, [pos], order_v[off], mask=wm)
            plsc.store_scatter(selv_v, [pos], svals_v[off], mask=wm)
            cnt = cnt + plsc.all_reduce_population_count(m)

        @pl.when(jnp.max(cnt) < KNB)
        def _fallback():
            rb = pl.multiple_of(row * NN, 8)
            pltpu.sync_copy(adj_hbm.at[pl.ds(rb, NN)], arow_v)
            seli_v[pl.ds(0, 16)] = zero16
            seli_v[pl.ds(16, 16)] = zero16
            selv_v[pl.ds(0, 16)] = neg16
            selv_v[pl.ds(16, 16)] = neg16

            def fb(t, cnt2):
                off = pl.ds(t * 16, 16)
                o16 = order_v[off]
                m2 = plsc.load_gather(arow_v, [o16]) > 0.0
                pos2 = cnt2 + plsc.cumsum(m2.astype(jnp.int32)) - 1
                wm2 = m2 & (pos2 < KNB)
                plsc.store_scatter(seli_v, [pos2], o16, mask=wm2)
                plsc.store_scatter(selv_v, [pos2], svals_v[off], mask=wm2)
                return cnt2 + plsc.all_reduce_population_count(m2)
            lax.fori_loop(0, NN // 16, fb, jnp.zeros((16,), jnp.int32))

        v0 = selv_v[pl.ds(0, 16)]
        v1 = selv_v[pl.ds(16, 16)]
        mxv = jnp.broadcast_to(jnp.maximum(jnp.max(v0), jnp.max(v1)), (16,))
        e0 = jnp.exp(v0 - mxv)
        e1 = jnp.exp(v1 - mxv)
        sv = jnp.broadcast_to(jnp.sum(e0) + jnp.sum(e1), (16,))
        wbuf_v[k, pl.ds(0, 16)] = e0 / sv
        wbuf_v[k, pl.ds(16, 16)] = e1 / sv
        ibuf_v[k, pl.ds(0, 16)] = seli_v[pl.ds(0, 16)]
        ibuf_v[k, pl.ds(16, 16)] = seli_v[pl.ds(16, 16)]

    for r in range(NRING):
        issue(r, r)

    def group(qq, carry):
        for r in range(NRING):
            pq = NRING * qq + r
            pltpu.make_async_copy(adj_hbm.at[idx_bufs[r]],
                                  g_bufs[r], sems[r]).wait()
            process(i0 + 2 * pq, 2 * pq, 0, idx_bufs[r], g_bufs[r])
            process(i0 + 2 * pq + 1, 2 * pq + 1, P0, idx_bufs[r], g_bufs[r])

            @pl.when(pq + NRING < PAIRS)
            def _next():
                issue(pq + NRING, r)
        return carry
    lax.fori_loop(0, PAIRS // NRING, group, 0)

    pltpu.sync_copy(wbuf_v, w_hbm.at[b, pl.ds(i0, RPW)])
    pltpu.sync_copy(ibuf_v, i_hbm.at[b, pl.ds(i0, RPW)])


def _sc_select(adj_flat, imp, rank):
    mesh = plsc.VectorSubcoreMesh(
        core_axis_name="c", subcore_axis_name="s",
        num_cores=NC, num_subcores=NS)
    fn = pl.kernel(
        _sc_select_body,
        out_type=(
            jax.ShapeDtypeStruct((NB, NN, KNB), jnp.float32),
            jax.ShapeDtypeStruct((NB, NN, KNB), jnp.int32),
        ),
        mesh=mesh,
        compiler_params=pltpu.CompilerParams(needs_layout_passes=False),
        scratch_types=[
            pltpu.VMEM((NN,), jnp.int32),
            pltpu.VMEM((NN,), jnp.float32),
            pltpu.VMEM((NN,), jnp.int32),
            pltpu.VMEM((NN,), jnp.float32),
            [pltpu.VMEM((2 * P0,), jnp.int32) for _ in range(NRING)],
            [pltpu.VMEM((2 * P0,), jnp.float32) for _ in range(NRING)],
            pltpu.VMEM((NN,), jnp.float32),
            pltpu.VMEM((KNB,), jnp.float32),
            pltpu.VMEM((KNB,), jnp.int32),
            pltpu.VMEM((RPW, KNB), jnp.float32),
            pltpu.VMEM((RPW, KNB), jnp.int32),
            [pltpu.SemaphoreType.DMA for _ in range(NRING)],
        ],
    )
    return fn(adj_flat, imp, rank)


def kernel(adj, features, attn_kernel):
    imp = _imp(features, attn_kernel)
    rank = _rank(imp.reshape(NB, NN, 1), imp)
    top_w, top_i = _sc_select(adj.reshape(NN * NN),
                              imp.reshape(NB, NN), rank.reshape(NB, NN))
    return (top_w, top_i)


# speedup vs baseline: 47.2103x; 1.2920x over previous
import functools

import jax
import jax.numpy as jnp
from jax import lax
from jax.experimental import pallas as pl
from jax.experimental.pallas import tpu as pltpu
from jax.experimental.pallas import tpu_sc as plsc

KNB = 32
NB, NN, FF = 2, 4096, 768
P0 = 32
NEG = -1000000000.0
NC, NS = 2, 16
NW = NC * NS
RPW = NB * NN // NW


RB = 1024


def _imp_body(feat_ref, ak_ref, imp_ref):
    f = feat_ref[0]
    ak = ak_ref[...]
    imp_col = jnp.dot(f, ak, preferred_element_type=jnp.float32)
    imp_ref[0, 0, :] = imp_col[:, 0]


def _imp(features, attn_kernel):
    return pl.pallas_call(
        _imp_body,
        grid=(NB,),
        in_specs=[
            pl.BlockSpec((1, NN, FF), lambda b: (b, 0, 0)),
            pl.BlockSpec((FF, 1), lambda b: (0, 0)),
        ],
        out_specs=pl.BlockSpec((1, 1, NN), lambda b: (b, 0, 0)),
        out_shape=jax.ShapeDtypeStruct((NB, 1, NN), jnp.float32),
    )(features, attn_kernel)


def _rank_body(ic_ref, ir_ref, rank_ref, acc_ref):
    t = pl.program_id(1)
    u = pl.program_id(2)
    col = ic_ref[0]
    row = ir_ref[0]

    ones = jnp.ones((RB, 1), jnp.float32)

    def put(hit):
        cnt = jnp.dot(hit.astype(jnp.float32), ones,
                      preferred_element_type=jnp.float32)[:, 0]

        @pl.when(u == 0)
        def _init():
            acc_ref[...] = cnt

        @pl.when(u != 0)
        def _acc():
            acc_ref[...] = acc_ref[...] + cnt

    @pl.when(u == t)
    def _d():
        jc = lax.broadcasted_iota(jnp.int32, (RB, 1), 0)
        jp = lax.broadcasted_iota(jnp.int32, (1, RB), 1)
        put((row > col) | ((row == col) & (jp < jc)))

    @pl.when(u < t)
    def _lo():
        put(row >= col)

    @pl.when(u > t)
    def _hi():
        put(row > col)

    @pl.when(u == NN // RB - 1)
    def _emit():
        rank_ref[0, 0, :] = acc_ref[...].astype(jnp.int32)


def _rank(imp_col3, imp_row3):
    return pl.pallas_call(
        _rank_body,
        grid=(NB, NN // RB, NN // RB),
        in_specs=[
            pl.BlockSpec((1, RB, 1), lambda b, t, u: (b, t, 0)),
            pl.BlockSpec((1, 1, RB), lambda b, t, u: (b, 0, u)),
        ],
        out_specs=pl.BlockSpec((1, 1, RB), lambda b, t, u: (b, 0, t)),
        out_shape=jax.ShapeDtypeStruct((NB, 1, NN), jnp.int32),
        scratch_shapes=[pltpu.VMEM((RB,), jnp.float32)],
    )(imp_col3, imp_row3)


NRING = 4
RPD = 4
QUADS = RPW // RPD


def _sc_select_body(adj_hbm, imp_hbm, rank_hbm, w_hbm, i_hbm,
                    rank_v, imp_v, order_v, svals_v,
                    idx_bufs, g_bufs, arow_v,
                    selv_v, seli_v, wbuf_v, ibuf_v, sems):
    cid = lax.axis_index("c")
    sid = lax.axis_index("s")
    b = cid
    i0 = sid * RPW

    pltpu.sync_copy(rank_hbm.at[b], rank_v)
    pltpu.sync_copy(imp_hbm.at[b], imp_v)

    def build(t, carry):
        off = pl.ds(t * 16, 16)
        r16 = rank_v[off]
        plsc.store_scatter(order_v, [r16], lax.iota(jnp.int32, 16) + t * 16)
        plsc.store_scatter(svals_v, [r16], imp_v[off])
        return carry
    lax.fori_loop(0, NN // 16, build, 0)

    zero16 = jnp.zeros((16,), jnp.int32)
    neg16 = jnp.full((16,), NEG, jnp.float32)

    def issue(pq, r):
        idx_ref, g_ref, sem = idx_bufs[r], g_bufs[r], sems[r]
        for h in range(RPD):
            ib = (i0 + RPD * pq + h) * NN
            for c in range(P0 // 16):
                idx_ref[pl.ds(h * P0 + c * 16, 16)] = (
                    order_v[pl.ds(c * 16, 16)] + ib)
        pltpu.async_copy(adj_hbm.at[idx_ref], g_ref, sem)

    def process(row, k, goff, idx_ref, g_ref):
        seli_v[pl.ds(0, 16)] = zero16
        seli_v[pl.ds(16, 16)] = zero16
        selv_v[pl.ds(0, 16)] = neg16
        selv_v[pl.ds(16, 16)] = neg16
        cnt = jnp.zeros((16,), jnp.int32)
        for c in range(P0 // 16):
            off = pl.ds(c * 16, 16)
            m = g_ref[pl.ds(goff + c * 16, 16)] > 0.0
            pos = cnt + plsc.cumsum(m.astype(jnp.int32)) - 1
            wm = m & (pos < KNB)
            plsc.store_scatter(seli_v, [pos], order_v[off], mask=wm)
            plsc.store_scatter(selv_v, [pos], svals_v[off], mask=wm)
            cnt = cnt + plsc.all_reduce_population_count(m)

        @pl.when(jnp.max(cnt) < KNB)
        def _fallback():
            rb = pl.multiple_of(row * NN, 8)
            pltpu.sync_copy(adj_hbm.at[pl.ds(rb, NN)], arow_v)
            seli_v[pl.ds(0, 16)] = zero16
            seli_v[pl.ds(16, 16)] = zero16
            selv_v[pl.ds(0, 16)] = neg16
            selv_v[pl.ds(16, 16)] = neg16

            def fb(t, cnt2):
                off = pl.ds(t * 16, 16)
                o16 = order_v[off]
                m2 = plsc.load_gather(arow_v, [o16]) > 0.0
                pos2 = cnt2 + plsc.cumsum(m2.astype(jnp.int32)) - 1
                wm2 = m2 & (pos2 < KNB)
                plsc.store_scatter(seli_v, [pos2], o16, mask=wm2)
                plsc.store_scatter(selv_v, [pos2], svals_v[off], mask=wm2)
                return cnt2 + plsc.all_reduce_population_count(m2)
            lax.fori_loop(0, NN // 16, fb, jnp.zeros((16,), jnp.int32))

        v0 = selv_v[pl.ds(0, 16)]
        v1 = selv_v[pl.ds(16, 16)]
        mxv = jnp.broadcast_to(jnp.maximum(jnp.max(v0), jnp.max(v1)), (16,))
        e0 = jnp.exp(v0 - mxv)
        e1 = jnp.exp(v1 - mxv)
        sv = jnp.broadcast_to(jnp.sum(e0) + jnp.sum(e1), (16,))
        wbuf_v[k, pl.ds(0, 16)] = e0 / sv
        wbuf_v[k, pl.ds(16, 16)] = e1 / sv
        ibuf_v[k, pl.ds(0, 16)] = seli_v[pl.ds(0, 16)]
        ibuf_v[k, pl.ds(16, 16)] = seli_v[pl.ds(16, 16)]

    for r in range(NRING):
        issue(r, r)

    def group(qq, carry):
        for r in range(NRING):
            pq = NRING * qq + r
            pltpu.make_async_copy(adj_hbm.at[idx_bufs[r]],
                                  g_bufs[r], sems[r]).wait()
            for h in range(RPD):
                process(i0 + RPD * pq + h, RPD * pq + h, h * P0,
                        idx_bufs[r], g_bufs[r])

            @pl.when(pq + NRING < QUADS)
            def _next():
                issue(pq + NRING, r)
        return carry
    lax.fori_loop(0, QUADS // NRING, group, 0)

    pltpu.sync_copy(wbuf_v, w_hbm.at[b, pl.ds(i0, RPW)])
    pltpu.sync_copy(ibuf_v, i_hbm.at[b, pl.ds(i0, RPW)])


def _sc_select(adj_flat, imp, rank):
    mesh = plsc.VectorSubcoreMesh(
        core_axis_name="c", subcore_axis_name="s",
        num_cores=NC, num_subcores=NS)
    fn = pl.kernel(
        _sc_select_body,
        out_type=(
            jax.ShapeDtypeStruct((NB, NN, KNB), jnp.float32),
            jax.ShapeDtypeStruct((NB, NN, KNB), jnp.int32),
        ),
        mesh=mesh,
        compiler_params=pltpu.CompilerParams(needs_layout_passes=False),
        scratch_types=[
            pltpu.VMEM((NN,), jnp.int32),
            pltpu.VMEM((NN,), jnp.float32),
            pltpu.VMEM((NN,), jnp.int32),
            pltpu.VMEM((NN,), jnp.float32),
            [pltpu.VMEM((RPD * P0,), jnp.int32) for _ in range(NRING)],
            [pltpu.VMEM((RPD * P0,), jnp.float32) for _ in range(NRING)],
            pltpu.VMEM((NN,), jnp.float32),
            pltpu.VMEM((KNB,), jnp.float32),
            pltpu.VMEM((KNB,), jnp.int32),
            pltpu.VMEM((RPW, KNB), jnp.float32),
            pltpu.VMEM((RPW, KNB), jnp.int32),
            [pltpu.SemaphoreType.DMA for _ in range(NRING)],
        ],
    )
    return fn(adj_flat, imp, rank)


def kernel(adj, features, attn_kernel):
    imp = _imp(features, attn_kernel)
    rank = _rank(imp.reshape(NB, NN, 1), imp)
    top_w, top_i = _sc_select(adj.reshape(NN * NN),
                              imp.reshape(NB, NN), rank.reshape(NB, NN))
    return (top_w, top_i)


# speedup vs baseline: 56.0543x; 1.1873x over previous
import functools

import jax
import jax.numpy as jnp
from jax import lax
from jax.experimental import pallas as pl
from jax.experimental.pallas import tpu as pltpu
from jax.experimental.pallas import tpu_sc as plsc

KNB = 32
NB, NN, FF = 2, 4096, 768
P0 = 32
NEG = -1000000000.0
NC, NS = 2, 16
NW = NC * NS
RPW = NB * NN // NW


RB = 1024


NIB = 4


def _imp_body(feat_ref, ak_ref, imp_ref):
    f = feat_ref[0]
    ak = ak_ref[...]
    imp_col = jnp.dot(f, ak, preferred_element_type=jnp.float32)
    imp_ref[0, 0, :] = imp_col[:, 0]


def _imp(features, attn_kernel):
    return pl.pallas_call(
        _imp_body,
        grid=(NB, NIB),
        in_specs=[
            pl.BlockSpec((1, NN // NIB, FF), lambda b, n: (b, n, 0)),
            pl.BlockSpec((FF, 1), lambda b, n: (0, 0)),
        ],
        out_specs=pl.BlockSpec((1, 1, NN // NIB), lambda b, n: (b, 0, n)),
        out_shape=jax.ShapeDtypeStruct((NB, 1, NN), jnp.float32),
    )(features, attn_kernel)


def _rank_body(ic_ref, ir_ref, rank_ref, acc_ref):
    t = pl.program_id(1)
    u = pl.program_id(2)
    col = ic_ref[...].reshape(RB, 1)
    row = ir_ref[0]

    ones = jnp.ones((RB, 1), jnp.float32)

    def put(hit):
        cnt = jnp.dot(hit.astype(jnp.float32), ones,
                      preferred_element_type=jnp.float32)[:, 0]

        @pl.when(u == 0)
        def _init():
            acc_ref[...] = cnt

        @pl.when(u != 0)
        def _acc():
            acc_ref[...] = acc_ref[...] + cnt

    @pl.when(u == t)
    def _d():
        jc = lax.broadcasted_iota(jnp.int32, (RB, 1), 0)
        jp = lax.broadcasted_iota(jnp.int32, (1, RB), 1)
        put((row > col) | ((row == col) & (jp < jc)))

    @pl.when(u < t)
    def _lo():
        put(row >= col)

    @pl.when(u > t)
    def _hi():
        put(row > col)

    @pl.when(u == NN // RB - 1)
    def _emit():
        rank_ref[0, 0, :] = acc_ref[...].astype(jnp.int32)


def _rank(imp3):
    return pl.pallas_call(
        _rank_body,
        grid=(NB, NN // RB, NN // RB),
        in_specs=[
            pl.BlockSpec((1, 1, RB), lambda b, t, u: (b, 0, t)),
            pl.BlockSpec((1, 1, RB), lambda b, t, u: (b, 0, u)),
        ],
        out_specs=pl.BlockSpec((1, 1, RB), lambda b, t, u: (b, 0, t)),
        out_shape=jax.ShapeDtypeStruct((NB, 1, NN), jnp.int32),
        scratch_shapes=[pltpu.VMEM((RB,), jnp.float32)],
    )(imp3, imp3)


NRING = 4
RPD = 4
QUADS = RPW // RPD


def _sc_select_body(adj_hbm, imp_hbm, rank_hbm, w_hbm, i_hbm,
                    rank_v, imp_v, order_v, svals_v,
                    idx_bufs, g_bufs, arow_v,
                    selv_v, seli_v, wbuf_v, ibuf_v, sems):
    cid = lax.axis_index("c")
    sid = lax.axis_index("s")
    b = cid
    i0 = sid * RPW

    pltpu.sync_copy(rank_hbm.at[b], rank_v)
    pltpu.sync_copy(imp_hbm.at[b], imp_v)

    def build(t, carry):
        off = pl.ds(t * 16, 16)
        r16 = rank_v[off]
        plsc.store_scatter(order_v, [r16], lax.iota(jnp.int32, 16) + t * 16)
        plsc.store_scatter(svals_v, [r16], imp_v[off])
        return carry
    lax.fori_loop(0, NN // 16, build, 0)

    zero16 = jnp.zeros((16,), jnp.int32)
    neg16 = jnp.full((16,), NEG, jnp.float32)

    o_c0 = order_v[pl.ds(0, 16)]
    o_c1 = order_v[pl.ds(16, 16)]
    vc0 = svals_v[pl.ds(0, 16)]
    vc1 = svals_v[pl.ds(16, 16)]
    mxc = jnp.broadcast_to(jnp.maximum(jnp.max(vc0), jnp.max(vc1)), (16,))
    ec0 = jnp.exp(vc0 - mxc)
    ec1 = jnp.exp(vc1 - mxc)
    svc = jnp.broadcast_to(jnp.sum(ec0) + jnp.sum(ec1), (16,))
    w_c0 = ec0 / svc
    w_c1 = ec1 / svc

    def issue(pq, r):
        idx_ref, g_ref, sem = idx_bufs[r], g_bufs[r], sems[r]
        for h in range(RPD):
            ib = (i0 + RPD * pq + h) * NN
            for c in range(P0 // 16):
                idx_ref[pl.ds(h * P0 + c * 16, 16)] = (
                    order_v[pl.ds(c * 16, 16)] + ib)
        pltpu.async_copy(adj_hbm.at[idx_ref], g_ref, sem)

    def process(row, k, goff, idx_ref, g_ref):
        m0 = g_ref[pl.ds(goff, 16)] > 0.0
        m1 = g_ref[pl.ds(goff + 16, 16)] > 0.0
        ok = jnp.all(m0) & jnp.all(m1)

        @pl.when(ok)
        def _fast():
            wbuf_v[k, pl.ds(0, 16)] = w_c0
            wbuf_v[k, pl.ds(16, 16)] = w_c1
            ibuf_v[k, pl.ds(0, 16)] = o_c0
            ibuf_v[k, pl.ds(16, 16)] = o_c1

        @pl.when(jnp.logical_not(ok))
        def _fallback():
            rb = pl.multiple_of(row * NN, 8)
            pltpu.sync_copy(adj_hbm.at[pl.ds(rb, NN)], arow_v)
            seli_v[pl.ds(0, 16)] = zero16
            seli_v[pl.ds(16, 16)] = zero16
            selv_v[pl.ds(0, 16)] = neg16
            selv_v[pl.ds(16, 16)] = neg16

            def fb(t, cnt2):
                off = pl.ds(t * 16, 16)
                o16 = order_v[off]
                m2 = plsc.load_gather(arow_v, [o16]) > 0.0
                pos2 = cnt2 + plsc.cumsum(m2.astype(jnp.int32)) - 1
                wm2 = m2 & (pos2 < KNB)
                plsc.store_scatter(seli_v, [pos2], o16, mask=wm2)
                plsc.store_scatter(selv_v, [pos2], svals_v[off], mask=wm2)
                return cnt2 + plsc.all_reduce_population_count(m2)
            lax.fori_loop(0, NN // 16, fb, jnp.zeros((16,), jnp.int32))

            v0 = selv_v[pl.ds(0, 16)]
            v1 = selv_v[pl.ds(16, 16)]
            mxv = jnp.broadcast_to(
                jnp.maximum(jnp.max(v0), jnp.max(v1)), (16,))
            e0 = jnp.exp(v0 - mxv)
            e1 = jnp.exp(v1 - mxv)
            sv = jnp.broadcast_to(jnp.sum(e0) + jnp.sum(e1), (16,))
            wbuf_v[k, pl.ds(0, 16)] = e0 / sv
            wbuf_v[k, pl.ds(16, 16)] = e1 / sv
            ibuf_v[k, pl.ds(0, 16)] = seli_v[pl.ds(0, 16)]
            ibuf_v[k, pl.ds(16, 16)] = seli_v[pl.ds(16, 16)]

    for r in range(NRING):
        issue(r, r)

    def group(qq, carry):
        for r in range(NRING):
            pq = NRING * qq + r
            pltpu.make_async_copy(adj_hbm.at[idx_bufs[r]],
                                  g_bufs[r], sems[r]).wait()
            for h in range(RPD):
                process(i0 + RPD * pq + h, RPD * pq + h, h * P0,
                        idx_bufs[r], g_bufs[r])

            @pl.when(pq + NRING < QUADS)
            def _next():
                issue(pq + NRING, r)
        return carry
    lax.fori_loop(0, QUADS // NRING, group, 0)

    pltpu.sync_copy(wbuf_v, w_hbm.at[b, pl.ds(i0, RPW)])
    pltpu.sync_copy(ibuf_v, i_hbm.at[b, pl.ds(i0, RPW)])


def _sc_select(adj_flat, imp, rank):
    mesh = plsc.VectorSubcoreMesh(
        core_axis_name="c", subcore_axis_name="s",
        num_cores=NC, num_subcores=NS)
    fn = pl.kernel(
        _sc_select_body,
        out_type=(
            jax.ShapeDtypeStruct((NB, NN, KNB), jnp.float32),
            jax.ShapeDtypeStruct((NB, NN, KNB), jnp.int32),
        ),
        mesh=mesh,
        compiler_params=pltpu.CompilerParams(needs_layout_passes=False),
        scratch_types=[
            pltpu.VMEM((NN,), jnp.int32),
            pltpu.VMEM((NN,), jnp.float32),
            pltpu.VMEM((NN,), jnp.int32),
            pltpu.VMEM((NN,), jnp.float32),
            [pltpu.VMEM((RPD * P0,), jnp.int32) for _ in range(NRING)],
            [pltpu.VMEM((RPD * P0,), jnp.float32) for _ in range(NRING)],
            pltpu.VMEM((NN,), jnp.float32),
            pltpu.VMEM((KNB,), jnp.float32),
            pltpu.VMEM((KNB,), jnp.int32),
            pltpu.VMEM((RPW, KNB), jnp.float32),
            pltpu.VMEM((RPW, KNB), jnp.int32),
            [pltpu.SemaphoreType.DMA for _ in range(NRING)],
        ],
    )
    return fn(adj_flat, imp, rank)


def kernel(adj, features, attn_kernel):
    imp = _imp(features, attn_kernel)
    rank = _rank(imp)
    top_w, top_i = _sc_select(adj.reshape(NN * NN),
                              imp.reshape(NB, NN), rank.reshape(NB, NN))
    return (top_w, top_i)


# speedup vs baseline: 59.0622x; 1.0537x over previous
import functools

import jax
import jax.numpy as jnp
from jax import lax
from jax.experimental import pallas as pl
from jax.experimental.pallas import tpu as pltpu
from jax.experimental.pallas import tpu_sc as plsc

KNB = 32
NB, NN, FF = 2, 4096, 768
P0 = 32
NEG = -1000000000.0
NC, NS = 2, 16
NW = NC * NS
RPW = NB * NN // NW


RB = 2048


NIB = 4


def _imp_body(feat_ref, ak_ref, imp_ref):
    f = feat_ref[0]
    ak = ak_ref[...]
    imp_col = jnp.dot(f, ak, preferred_element_type=jnp.float32)
    imp_ref[0, 0, :] = imp_col[:, 0]


def _imp(features, attn_kernel):
    return pl.pallas_call(
        _imp_body,
        grid=(NB, NIB),
        in_specs=[
            pl.BlockSpec((1, NN // NIB, FF), lambda b, n: (b, n, 0)),
            pl.BlockSpec((FF, 1), lambda b, n: (0, 0)),
        ],
        out_specs=pl.BlockSpec((1, 1, NN // NIB), lambda b, n: (b, 0, n)),
        out_shape=jax.ShapeDtypeStruct((NB, 1, NN), jnp.float32),
    )(features, attn_kernel)


def _rank_body(ic_ref, ir_ref, rank_ref, acc_ref):
    t = pl.program_id(1)
    u = pl.program_id(2)
    col = ic_ref[...].reshape(RB, 1)
    row = ir_ref[0]

    ones = jnp.ones((RB, 1), jnp.float32)

    def put(hit):
        cnt = jnp.dot(hit.astype(jnp.float32), ones,
                      preferred_element_type=jnp.float32)[:, 0]

        @pl.when(u == 0)
        def _init():
            acc_ref[...] = cnt

        @pl.when(u != 0)
        def _acc():
            acc_ref[...] = acc_ref[...] + cnt

    @pl.when(u == t)
    def _d():
        jc = lax.broadcasted_iota(jnp.int32, (RB, 1), 0)
        jp = lax.broadcasted_iota(jnp.int32, (1, RB), 1)
        put((row > col) | ((row == col) & (jp < jc)))

    @pl.when(u < t)
    def _lo():
        put(row >= col)

    @pl.when(u > t)
    def _hi():
        put(row > col)

    @pl.when(u == NN // RB - 1)
    def _emit():
        rank_ref[0, 0, :] = acc_ref[...].astype(jnp.int32)


def _rank(imp3):
    return pl.pallas_call(
        _rank_body,
        grid=(NB, NN // RB, NN // RB),
        in_specs=[
            pl.BlockSpec((1, 1, RB), lambda b, t, u: (b, 0, t)),
            pl.BlockSpec((1, 1, RB), lambda b, t, u: (b, 0, u)),
        ],
        out_specs=pl.BlockSpec((1, 1, RB), lambda b, t, u: (b, 0, t)),
        out_shape=jax.ShapeDtypeStruct((NB, 1, NN), jnp.int32),
        scratch_shapes=[pltpu.VMEM((RB,), jnp.float32)],
    )(imp3, imp3)


NRING = 8
RPD = 4
QUADS = RPW // RPD


def _sc_select_body(adj_hbm, imp_hbm, rank_hbm, w_hbm, i_hbm,
                    rank_v, imp_v, order_v, svals_v,
                    idx_bufs, g_bufs, arow_v,
                    selv_v, seli_v, wbuf_v, ibuf_v, sems):
    cid = lax.axis_index("c")
    sid = lax.axis_index("s")
    b = cid
    i0 = sid * RPW

    pltpu.sync_copy(rank_hbm.at[b], rank_v)
    pltpu.sync_copy(imp_hbm.at[b], imp_v)

    def build(t, carry):
        off = pl.ds(t * 16, 16)
        r16 = rank_v[off]
        plsc.store_scatter(order_v, [r16], lax.iota(jnp.int32, 16) + t * 16)
        plsc.store_scatter(svals_v, [r16], imp_v[off])
        return carry
    lax.fori_loop(0, NN // 16, build, 0)

    zero16 = jnp.zeros((16,), jnp.int32)
    neg16 = jnp.full((16,), NEG, jnp.float32)

    o_c0 = order_v[pl.ds(0, 16)]
    o_c1 = order_v[pl.ds(16, 16)]
    vc0 = svals_v[pl.ds(0, 16)]
    vc1 = svals_v[pl.ds(16, 16)]
    mxc = jnp.broadcast_to(jnp.maximum(jnp.max(vc0), jnp.max(vc1)), (16,))
    ec0 = jnp.exp(vc0 - mxc)
    ec1 = jnp.exp(vc1 - mxc)
    svc = jnp.broadcast_to(jnp.sum(ec0) + jnp.sum(ec1), (16,))
    w_c0 = ec0 / svc
    w_c1 = ec1 / svc

    def issue(pq, r):
        idx_ref, g_ref, sem = idx_bufs[r], g_bufs[r], sems[r]
        for h in range(RPD):
            ib = (i0 + RPD * pq + h) * NN
            for c in range(P0 // 16):
                idx_ref[pl.ds(h * P0 + c * 16, 16)] = (
                    order_v[pl.ds(c * 16, 16)] + ib)
        pltpu.async_copy(adj_hbm.at[idx_ref], g_ref, sem)

    def process(row, k, goff, idx_ref, g_ref):
        m0 = g_ref[pl.ds(goff, 16)] > 0.0
        m1 = g_ref[pl.ds(goff + 16, 16)] > 0.0
        ok = jnp.all(m0) & jnp.all(m1)

        @pl.when(ok)
        def _fast():
            wbuf_v[k, pl.ds(0, 16)] = w_c0
            wbuf_v[k, pl.ds(16, 16)] = w_c1
            ibuf_v[k, pl.ds(0, 16)] = o_c0
            ibuf_v[k, pl.ds(16, 16)] = o_c1

        @pl.when(jnp.logical_not(ok))
        def _fallback():
            rb = pl.multiple_of(row * NN, 8)
            pltpu.sync_copy(adj_hbm.at[pl.ds(rb, NN)], arow_v)
            seli_v[pl.ds(0, 16)] = zero16
            seli_v[pl.ds(16, 16)] = zero16
            selv_v[pl.ds(0, 16)] = neg16
            selv_v[pl.ds(16, 16)] = neg16

            def fb(t, cnt2):
                off = pl.ds(t * 16, 16)
                o16 = order_v[off]
                m2 = plsc.load_gather(arow_v, [o16]) > 0.0
                pos2 = cnt2 + plsc.cumsum(m2.astype(jnp.int32)) - 1
                wm2 = m2 & (pos2 < KNB)
                plsc.store_scatter(seli_v, [pos2], o16, mask=wm2)
                plsc.store_scatter(selv_v, [pos2], svals_v[off], mask=wm2)
                return cnt2 + plsc.all_reduce_population_count(m2)
            lax.fori_loop(0, NN // 16, fb, jnp.zeros((16,), jnp.int32))

            v0 = selv_v[pl.ds(0, 16)]
            v1 = selv_v[pl.ds(16, 16)]
            mxv = jnp.broadcast_to(
                jnp.maximum(jnp.max(v0), jnp.max(v1)), (16,))
            e0 = jnp.exp(v0 - mxv)
            e1 = jnp.exp(v1 - mxv)
            sv = jnp.broadcast_to(jnp.sum(e0) + jnp.sum(e1), (16,))
            wbuf_v[k, pl.ds(0, 16)] = e0 / sv
            wbuf_v[k, pl.ds(16, 16)] = e1 / sv
            ibuf_v[k, pl.ds(0, 16)] = seli_v[pl.ds(0, 16)]
            ibuf_v[k, pl.ds(16, 16)] = seli_v[pl.ds(16, 16)]

    for r in range(NRING):
        issue(r, r)

    def group(qq, carry):
        for r in range(NRING):
            pq = NRING * qq + r
            pltpu.make_async_copy(adj_hbm.at[idx_bufs[r]],
                                  g_bufs[r], sems[r]).wait()
            for h in range(RPD):
                process(i0 + RPD * pq + h, RPD * pq + h, h * P0,
                        idx_bufs[r], g_bufs[r])

            @pl.when(pq + NRING < QUADS)
            def _next():
                issue(pq + NRING, r)
        return carry
    lax.fori_loop(0, QUADS // NRING, group, 0)

    pltpu.sync_copy(wbuf_v, w_hbm.at[b, pl.ds(i0, RPW)])
    pltpu.sync_copy(ibuf_v, i_hbm.at[b, pl.ds(i0, RPW)])


def _sc_select(adj_flat, imp, rank):
    mesh = plsc.VectorSubcoreMesh(
        core_axis_name="c", subcore_axis_name="s",
        num_cores=NC, num_subcores=NS)
    fn = pl.kernel(
        _sc_select_body,
        out_type=(
            jax.ShapeDtypeStruct((NB, NN, KNB), jnp.float32),
            jax.ShapeDtypeStruct((NB, NN, KNB), jnp.int32),
        ),
        mesh=mesh,
        compiler_params=pltpu.CompilerParams(needs_layout_passes=False),
        scratch_types=[
            pltpu.VMEM((NN,), jnp.int32),
            pltpu.VMEM((NN,), jnp.float32),
            pltpu.VMEM((NN,), jnp.int32),
            pltpu.VMEM((NN,), jnp.float32),
            [pltpu.VMEM((RPD * P0,), jnp.int32) for _ in range(NRING)],
            [pltpu.VMEM((RPD * P0,), jnp.float32) for _ in range(NRING)],
            pltpu.VMEM((NN,), jnp.float32),
            pltpu.VMEM((KNB,), jnp.float32),
            pltpu.VMEM((KNB,), jnp.int32),
            pltpu.VMEM((RPW, KNB), jnp.float32),
            pltpu.VMEM((RPW, KNB), jnp.int32),
            [pltpu.SemaphoreType.DMA for _ in range(NRING)],
        ],
    )
    return fn(adj_flat, imp, rank)


def kernel(adj, features, attn_kernel):
    imp = _imp(features, attn_kernel)
    rank = _rank(imp)
    top_w, top_i = _sc_select(adj.reshape(NN * NN),
                              imp.reshape(NB, NN), rank.reshape(NB, NN))
    return (top_w, top_i)


# speedup vs baseline: 60.0148x; 1.0161x over previous
import functools

import jax
import jax.numpy as jnp
from jax import lax
from jax.experimental import pallas as pl
from jax.experimental.pallas import tpu as pltpu
from jax.experimental.pallas import tpu_sc as plsc

KNB = 32
NB, NN, FF = 2, 4096, 768
P0 = 32
NEG = -1000000000.0
NC, NS = 2, 16
NW = NC * NS
RPW = NB * NN // NW


RB = 2048


NIB = 4


def _imp_body(feat_ref, ak_ref, imp_ref):
    f = feat_ref[0]
    ak = ak_ref[...]
    imp_col = jnp.dot(f, ak, preferred_element_type=jnp.float32)
    imp_ref[0, 0, :] = imp_col[:, 0]


def _imp(features, attn_kernel):
    return pl.pallas_call(
        _imp_body,
        grid=(NB, NIB),
        in_specs=[
            pl.BlockSpec((1, NN // NIB, FF), lambda b, n: (b, n, 0)),
            pl.BlockSpec((FF, 1), lambda b, n: (0, 0)),
        ],
        out_specs=pl.BlockSpec((1, 1, NN // NIB), lambda b, n: (b, 0, n)),
        out_shape=jax.ShapeDtypeStruct((NB, 1, NN), jnp.float32),
    )(features, attn_kernel)


def _rank_body(ic_ref, ir_ref, rank_ref, acc_ref):
    t = pl.program_id(1)
    u = pl.program_id(2)
    col = ic_ref[...].reshape(RB, 1)
    row = ir_ref[0]

    ones = jnp.ones((RB, 1), jnp.float32)

    def put(hit):
        cnt = jnp.sum(hit.astype(jnp.float32), axis=1)

        @pl.when(u == 0)
        def _init():
            acc_ref[...] = cnt

        @pl.when(u != 0)
        def _acc():
            acc_ref[...] = acc_ref[...] + cnt

    @pl.when(u == t)
    def _d():
        jc = lax.broadcasted_iota(jnp.int32, (RB, 1), 0)
        jp = lax.broadcasted_iota(jnp.int32, (1, RB), 1)
        put((row > col) | ((row == col) & (jp < jc)))

    @pl.when(u < t)
    def _lo():
        put(row >= col)

    @pl.when(u > t)
    def _hi():
        put(row > col)

    @pl.when(u == NN // RB - 1)
    def _emit():
        rank_ref[0, 0, :] = acc_ref[...].astype(jnp.int32)


def _rank(imp3):
    return pl.pallas_call(
        _rank_body,
        grid=(NB, NN // RB, NN // RB),
        in_specs=[
            pl.BlockSpec((1, 1, RB), lambda b, t, u: (b, 0, t)),
            pl.BlockSpec((1, 1, RB), lambda b, t, u: (b, 0, u)),
        ],
        out_specs=pl.BlockSpec((1, 1, RB), lambda b, t, u: (b, 0, t)),
        out_shape=jax.ShapeDtypeStruct((NB, 1, NN), jnp.int32),
        scratch_shapes=[pltpu.VMEM((RB,), jnp.float32)],
    )(imp3, imp3)


NRING = 8
RPD = 4
QUADS = RPW // RPD


def _sc_select_body(adj_hbm, imp_hbm, rank_hbm, w_hbm, i_hbm,
                    rank_v, imp_v, order_v, svals_v,
                    idx_bufs, g_bufs, arow_v,
                    selv_v, seli_v, wbuf_v, ibuf_v, sems):
    cid = lax.axis_index("c")
    sid = lax.axis_index("s")
    b = cid
    i0 = sid * RPW

    pltpu.sync_copy(rank_hbm.at[b], rank_v)
    pltpu.sync_copy(imp_hbm.at[b], imp_v)

    def build(t, carry):
        off = pl.ds(t * 16, 16)
        r16 = rank_v[off]
        plsc.store_scatter(order_v, [r16], lax.iota(jnp.int32, 16) + t * 16)
        plsc.store_scatter(svals_v, [r16], imp_v[off])
        return carry
    lax.fori_loop(0, NN // 16, build, 0)

    zero16 = jnp.zeros((16,), jnp.int32)
    neg16 = jnp.full((16,), NEG, jnp.float32)

    o_c0 = order_v[pl.ds(0, 16)]
    o_c1 = order_v[pl.ds(16, 16)]
    vc0 = svals_v[pl.ds(0, 16)]
    vc1 = svals_v[pl.ds(16, 16)]
    mxc = jnp.broadcast_to(jnp.maximum(jnp.max(vc0), jnp.max(vc1)), (16,))
    ec0 = jnp.exp(vc0 - mxc)
    ec1 = jnp.exp(vc1 - mxc)
    svc = jnp.broadcast_to(jnp.sum(ec0) + jnp.sum(ec1), (16,))
    w_c0 = ec0 / svc
    w_c1 = ec1 / svc

    def issue(pq, r):
        idx_ref, g_ref, sem = idx_bufs[r], g_bufs[r], sems[r]
        for h in range(RPD):
            ib = (i0 + RPD * pq + h) * NN
            for c in range(P0 // 16):
                idx_ref[pl.ds(h * P0 + c * 16, 16)] = (
                    order_v[pl.ds(c * 16, 16)] + ib)
        pltpu.async_copy(adj_hbm.at[idx_ref], g_ref, sem)

    def process(row, k, goff, idx_ref, g_ref):
        m0 = g_ref[pl.ds(goff, 16)] > 0.0
        m1 = g_ref[pl.ds(goff + 16, 16)] > 0.0
        ok = jnp.all(m0) & jnp.all(m1)

        @pl.when(ok)
        def _fast():
            wbuf_v[k, pl.ds(0, 16)] = w_c0
            wbuf_v[k, pl.ds(16, 16)] = w_c1
            ibuf_v[k, pl.ds(0, 16)] = o_c0
            ibuf_v[k, pl.ds(16, 16)] = o_c1

        @pl.when(jnp.logical_not(ok))
        def _fallback():
            rb = pl.multiple_of(row * NN, 8)
            pltpu.sync_copy(adj_hbm.at[pl.ds(rb, NN)], arow_v)
            seli_v[pl.ds(0, 16)] = zero16
            seli_v[pl.ds(16, 16)] = zero16
            selv_v[pl.ds(0, 16)] = neg16
            selv_v[pl.ds(16, 16)] = neg16

            def fb(t, cnt2):
                off = pl.ds(t * 16, 16)
                o16 = order_v[off]
                m2 = plsc.load_gather(arow_v, [o16]) > 0.0
                pos2 = cnt2 + plsc.cumsum(m2.astype(jnp.int32)) - 1
                wm2 = m2 & (pos2 < KNB)
                plsc.store_scatter(seli_v, [pos2], o16, mask=wm2)
                plsc.store_scatter(selv_v, [pos2], svals_v[off], mask=wm2)
                return cnt2 + plsc.all_reduce_population_count(m2)
            lax.fori_loop(0, NN // 16, fb, jnp.zeros((16,), jnp.int32))

            v0 = selv_v[pl.ds(0, 16)]
            v1 = selv_v[pl.ds(16, 16)]
            mxv = jnp.broadcast_to(
                jnp.maximum(jnp.max(v0), jnp.max(v1)), (16,))
            e0 = jnp.exp(v0 - mxv)
            e1 = jnp.exp(v1 - mxv)
            sv = jnp.broadcast_to(jnp.sum(e0) + jnp.sum(e1), (16,))
            wbuf_v[k, pl.ds(0, 16)] = e0 / sv
            wbuf_v[k, pl.ds(16, 16)] = e1 / sv
            ibuf_v[k, pl.ds(0, 16)] = seli_v[pl.ds(0, 16)]
            ibuf_v[k, pl.ds(16, 16)] = seli_v[pl.ds(16, 16)]

    for r in range(NRING):
        issue(r, r)

    def group(qq, carry):
        for r in range(NRING):
            pq = NRING * qq + r
            pltpu.make_async_copy(adj_hbm.at[idx_bufs[r]],
                                  g_bufs[r], sems[r]).wait()
            for h in range(RPD):
                process(i0 + RPD * pq + h, RPD * pq + h, h * P0,
                        idx_bufs[r], g_bufs[r])

            @pl.when(pq + NRING < QUADS)
            def _next():
                issue(pq + NRING, r)
        return carry
    lax.fori_loop(0, QUADS // NRING, group, 0)

    pltpu.sync_copy(wbuf_v, w_hbm.at[b, pl.ds(i0, RPW)])
    pltpu.sync_copy(ibuf_v, i_hbm.at[b, pl.ds(i0, RPW)])


def _sc_select(adj_flat, imp, rank):
    mesh = plsc.VectorSubcoreMesh(
        core_axis_name="c", subcore_axis_name="s",
        num_cores=NC, num_subcores=NS)
    fn = pl.kernel(
        _sc_select_body,
        out_type=(
            jax.ShapeDtypeStruct((NB, NN, KNB), jnp.float32),
            jax.ShapeDtypeStruct((NB, NN, KNB), jnp.int32),
        ),
        mesh=mesh,
        compiler_params=pltpu.CompilerParams(needs_layout_passes=False),
        scratch_types=[
            pltpu.VMEM((NN,), jnp.int32),
            pltpu.VMEM((NN,), jnp.float32),
            pltpu.VMEM((NN,), jnp.int32),
            pltpu.VMEM((NN,), jnp.float32),
            [pltpu.VMEM((RPD * P0,), jnp.int32) for _ in range(NRING)],
            [pltpu.VMEM((RPD * P0,), jnp.float32) for _ in range(NRING)],
            pltpu.VMEM((NN,), jnp.float32),
            pltpu.VMEM((KNB,), jnp.float32),
            pltpu.VMEM((KNB,), jnp.int32),
            pltpu.VMEM((RPW, KNB), jnp.float32),
            pltpu.VMEM((RPW, KNB), jnp.int32),
            [pltpu.SemaphoreType.DMA for _ in range(NRING)],
        ],
    )
    return fn(adj_flat, imp, rank)


def kernel(adj, features, attn_kernel):
    imp = _imp(features, attn_kernel)
    rank = _rank(imp)
    top_w, top_i = _sc_select(adj.reshape(NN * NN),
                              imp.reshape(NB, NN), rank.reshape(NB, NN))
    return (top_w, top_i)
